# baseline (speedup 1.0000x reference)
import functools

import jax
import jax.numpy as jnp
from jax import lax
from jax.experimental import pallas as pl
from jax.experimental.pallas import tpu as pltpu
from jax.experimental.pallas import tpu_sc as plsc

_NV = 1000
_NF = 26
_CD = 32
_B = 1024
_NW = 32
_RPW = _B // _NW
_ROWSTRIDE = _NF * (_NV + 1)


def _sc_body(x_hbm, i_hbm, s_hbm, w_hbm, b_hbm, wmean_hbm, brep_hbm,
             x_v, i_v, s_v, idx_v, ipos_v, rows_v, brow_v, wm_v, brep_v,
             sem):
    wid = lax.axis_index("s") * 2 + lax.axis_index("c")
    base = wid * _RPW

    pltpu.sync_copy(x_hbm.at[pl.ds(base * _NF, _RPW * _NF)], x_v)
    pltpu.sync_copy(i_hbm.at[pl.ds(base * _NF, _RPW * _NF)], i_v)
    pltpu.sync_copy(s_hbm.at[pl.ds(base * _NF, _RPW * _NF)], s_v)

    lanes = lax.iota(jnp.int32, 16)
    for g in range(_RPW // 16):
        roff = (jnp.full((16,), g * 16, jnp.int32) + lanes) * _NF
        best_v = plsc.load_gather(i_v, [roff])
        best_i = jnp.zeros((16,), jnp.int32)
        for k in range(1, _NF):
            kv = jnp.full((16,), k, jnp.int32)
            v = plsc.load_gather(i_v, [roff + k])
            m = v > best_v
            best_i = jnp.where(m, kv, best_i)
            best_v = jnp.where(m, v, best_v)
        ipos_v[pl.ds(g * 16, 16)] = best_i
        ibase = best_i * _ROWSTRIDE
        for k in range(_NF):
            xv = plsc.load_gather(x_v, [roff + k])
            sv = plsc.load_gather(s_v, [roff + k])
            col = xv * sv + (1 - sv) * _NV
            idx_v[k, pl.ds(g * 16, 16)] = ibase + k * (_NV + 1) + col

    copies = [pltpu.async_copy(w_hbm.at[idx_v.at[k]], rows_v.at[k], sem)
              for k in range(_NF)]
    bcopy = pltpu.async_copy(b_hbm.at[ipos_v], brow_v, sem)
    for c in copies:
        c.wait()
    bcopy.wait()

    inv = jnp.float32(1.0 / _NF)

    @pl.loop(0, _RPW)
    def _(r):
        acc0 = jnp.zeros((16,), jnp.float32)
        acc1 = jnp.zeros((16,), jnp.float32)
        for k in range(_NF):
            acc0 = acc0 + rows_v[k, r, pl.ds(0, 16)]
            acc1 = acc1 + rows_v[k, r, pl.ds(16, 16)]
        wm_v[r, pl.ds(0, 16)] = acc0 * inv
        wm_v[r, pl.ds(16, 16)] = acc1 * inv
        b0 = brow_v[r, pl.ds(0, 16)]
        b1 = brow_v[r, pl.ds(16, 16)]
        for t in range(4):
            brep_v[r, pl.ds(t * 32, 16)] = b0
            brep_v[r, pl.ds(t * 32 + 16, 16)] = b1

    pltpu.sync_copy(wm_v, wmean_hbm.at[pl.ds(base, _RPW)])
    pltpu.sync_copy(brep_v, brep_hbm.at[pl.ds(base, _RPW)])


_sc_call_cache = []


def _sc_call(*argv):
    if not _sc_call_cache:
        _sc_call_cache.append(functools.partial(
            pl.kernel,
            out_type=(
                jax.ShapeDtypeStruct((_B, _CD), jnp.float32),
                jax.ShapeDtypeStruct((_B, 128), jnp.float32),
            ),
            mesh=plsc.VectorSubcoreMesh(core_axis_name="c",
                                        subcore_axis_name="s"),
            compiler_params=pltpu.CompilerParams(needs_layout_passes=False,
                                                 use_tc_tiling_on_sc=False),
            scratch_types=[
                pltpu.VMEM((_RPW * _NF,), jnp.int32),
                pltpu.VMEM((_RPW * _NF,), jnp.float32),
                pltpu.VMEM((_RPW * _NF,), jnp.int32),
                pltpu.VMEM((_NF, _RPW), jnp.int32),
                pltpu.VMEM((_RPW,), jnp.int32),
                pltpu.VMEM((_NF, _RPW, _CD), jnp.float32),
                pltpu.VMEM((_RPW, _CD), jnp.float32),
                pltpu.VMEM((_RPW, _CD), jnp.float32),
                pltpu.VMEM((_RPW, 128), jnp.float32),
                pltpu.SemaphoreType.DMA,
            ],
        )(_sc_body))
    return _sc_call_cache[0](*argv)


def _tc_body(wm_ref, br_ref, out_ref):
    out_ref[...] = wm_ref[...][None, :, :] + br_ref[...][:, None, :]


_BI = 16


def _tc_call(wm_r, brep):
    return pl.pallas_call(
        _tc_body,
        grid=(_B // _BI,),
        in_specs=[
            pl.BlockSpec((_B * _CD // 128, 128), lambda i: (0, 0)),
            pl.BlockSpec((_BI, 128), lambda i: (i, 0)),
        ],
        out_specs=pl.BlockSpec((_BI, _B * _CD // 128, 128),
                               lambda i: (i, 0, 0)),
        out_shape=jax.ShapeDtypeStruct((_B, _B * _CD // 128, 128),
                                       jnp.float32),
    )(wm_r, brep)


@jax.jit
def kernel(X, I, S, weights, bias):
    X = X.astype(jnp.int32)
    S = S.astype(jnp.int32)
    wmean, brep = _sc_call(X.reshape(-1), I.reshape(-1), S.reshape(-1),
                           weights, bias)
    wm_r = wmean.reshape(_B * _CD // 128, 128)
    out = _tc_call(wm_r, brep)
    return out.reshape(_B, _B, _CD)


# speedup vs baseline: 1.3067x; 1.3067x over previous
import functools

import jax
import jax.numpy as jnp
from jax import lax
from jax.experimental import pallas as pl
from jax.experimental.pallas import tpu as pltpu
from jax.experimental.pallas import tpu_sc as plsc

_NV = 1000
_NF = 26
_CD = 32
_B = 1024
_NW = 32
_RPW = _B // _NW
_ROWSTRIDE = _NF * (_NV + 1)
_NROW = (_NV + 1) * _NF * _NF
_NPACK = _NROW // 4


_CHM = 2048


def _repack_body(in_ref, out_ref):
    t = in_ref[...].reshape(_CD, _CHM)
    tt3 = t.T.reshape(_CHM // 4, 4, _CD)
    for j in range(4):
        out_ref[:, pl.ds(j * _CD, _CD)] = tt3[:, j, :]


def _repack(wt3):
    n_full = _NROW // _CHM
    rem = _NROW - n_full * _CHM
    main = pl.pallas_call(
        _repack_body,
        grid=(n_full,),
        in_specs=[pl.BlockSpec((4, 8, _CHM), lambda i: (0, 0, i))],
        out_specs=pl.BlockSpec((_CHM // 4, 128), lambda i: (i, 0)),
        out_shape=jax.ShapeDtypeStruct((n_full * _CHM // 4, 128),
                                       jnp.float32),
    )(wt3)
    def _rem_body(in_ref, out_ref):
        t = in_ref[...].reshape(_CD, rem)
        tt3 = t.T.reshape(rem // 4, 4, _CD)
        for j in range(4):
            out_ref[:, pl.ds(j * _CD, _CD)] = tt3[:, j, :]
    tail = pl.pallas_call(
        _rem_body,
        in_specs=[pl.BlockSpec((4, 8, rem), lambda: (0, 0, 0))],
        out_specs=pl.BlockSpec((rem // 4, 128), lambda: (0, 0)),
        grid=(),
        out_shape=jax.ShapeDtypeStruct((rem // 4, 128), jnp.float32),
    )(jax.lax.slice(wt3, (0, 0, n_full * _CHM), (4, 8, _NROW)))
    return jnp.concatenate([main, tail], axis=0)


def _sc_body(x_t, i_t, s_t, wlin, b_hbm, wm_hbm, bv_hbm,
             xT_v, iT_v, sT_v, bias_v, idx_v, off_v, ipos_v, rows_v,
             wm_v, bv_v, sem):
    wid = lax.axis_index("s") * 2 + lax.axis_index("c")
    base = wid * _RPW
    slab = pl.multiple_of((wid // 4) * 128, 128)
    q = (wid % 4) * _RPW

    pltpu.sync_copy(x_t.at[:, pl.ds(slab, 128)], xT_v)
    pltpu.sync_copy(i_t.at[:, pl.ds(slab, 128)], iT_v)
    pltpu.sync_copy(s_t.at[:, pl.ds(slab, 128)], sT_v)
    pltpu.sync_copy(b_hbm, bias_v)

    lanes = lax.iota(jnp.int32, 16)
    for g in range(_RPW // 16):
        sl = pl.ds(q + g * 16, 16)
        best_v = iT_v[0, sl]
        best_i = jnp.zeros((16,), jnp.int32)
        for k in range(1, _NF):
            v = iT_v[k, sl]
            m = v > best_v
            best_i = jnp.where(m, jnp.full((16,), k, jnp.int32), best_i)
            best_v = jnp.where(m, v, best_v)
        ipos_v[pl.ds(g * 16, 16)] = best_i
        ibase = best_i * _ROWSTRIDE
        rows16 = jnp.full((16,), g * 16, jnp.int32) + lanes
        for k in range(_NF):
            xv = xT_v[k, sl]
            sv = sT_v[k, sl]
            col = xv * sv + (1 - sv) * _NV
            ridx = ibase + k * (_NV + 1) + col
            idx_v[k, pl.ds(g * 16, 16)] = lax.shift_right_logical(ridx, 2)
            off_v[k, pl.ds(g * 16, 16)] = (
                lax.shift_left(jnp.bitwise_and(ridx, 3), 5))
        for c in range(_CD):
            bvv = plsc.load_gather(bias_v,
                                   [best_i, jnp.full((16,), c, jnp.int32)])
            plsc.store_scatter(bv_v, [rows16, jnp.full((16,), c, jnp.int32)],
                               bvv)

    inv = jnp.float32(1.0 / _NF)
    for g in range(_RPW // 16):
        copies = [pltpu.async_copy(wlin.at[idx_v.at[k, pl.ds(g * 16, 16)]],
                                   rows_v.at[k], sem)
                  for k in range(_NF)]
        for cp in copies:
            cp.wait()

        gsl = pl.ds(g * 16, 16)
        grows = jnp.full((16,), g * 16, jnp.int32) + lanes

        @pl.loop(0, _CD)
        def _(c, _g=g, _sl=gsl, _rows=grows):
            acc = jnp.zeros((16,), jnp.float32)
            for k in range(_NF):
                kv = jnp.full((16,), k, jnp.int32)
                acc = acc + plsc.load_gather(
                    rows_v, [kv, lanes, off_v[k, _sl] + c])
            plsc.store_scatter(wm_v, [_rows, jnp.zeros((16,), jnp.int32) + c],
                               acc * inv)

    pltpu.sync_copy(wm_v, wm_hbm.at[pl.ds(base, _RPW)])
    pltpu.sync_copy(bv_v, bv_hbm.at[pl.ds(base, _RPW)])


_sc_call_cache = []


def _sc_call(*argv):
    if not _sc_call_cache:
        _sc_call_cache.append(functools.partial(
            pl.kernel,
            out_type=(
                jax.ShapeDtypeStruct((_B, _CD), jnp.float32),
                jax.ShapeDtypeStruct((_B, _CD), jnp.float32),
            ),
            mesh=plsc.VectorSubcoreMesh(core_axis_name="c",
                                        subcore_axis_name="s"),
            compiler_params=pltpu.CompilerParams(needs_layout_passes=False,
                                                 use_tc_tiling_on_sc=True),
            scratch_types=[
                pltpu.VMEM((_NF, 128), jnp.int32),
                pltpu.VMEM((_NF, 128), jnp.float32),
                pltpu.VMEM((_NF, 128), jnp.int32),
                pltpu.VMEM((_NF, _CD), jnp.float32),
                pltpu.VMEM((_NF, _RPW), jnp.int32),
                pltpu.VMEM((_NF, _RPW), jnp.int32),
                pltpu.VMEM((_RPW,), jnp.int32),
                pltpu.VMEM((_NF, 16, 128), jnp.float32),
                pltpu.VMEM((_RPW, _CD), jnp.float32),
                pltpu.VMEM((_RPW, _CD), jnp.float32),
                pltpu.SemaphoreType.DMA,
            ],
        )(_sc_body))
    return _sc_call_cache[0](*argv)


def _tc_body(wm_ref, bv_ref, out_ref):
    out_ref[...] = wm_ref[...][None, :, :] + bv_ref[...][:, :, None]


_BI = 16


def _tc_call(wmT, bv):
    return pl.pallas_call(
        _tc_body,
        grid=(_B // _BI,),
        in_specs=[
            pl.BlockSpec((_CD, _B), lambda i: (0, 0)),
            pl.BlockSpec((_BI, _CD), lambda i: (i, 0)),
        ],
        out_specs=pl.BlockSpec((_BI, _CD, _B), lambda i: (i, 0, 0)),
        out_shape=jax.ShapeDtypeStruct((_B, _CD, _B), jnp.float32),
    )(wmT, bv)


@jax.jit
def kernel(X, I, S, weights, bias):
    X = X.astype(jnp.int32)
    S = S.astype(jnp.int32)
    wt3 = weights.T.reshape(4, 8, _NROW)
    wlin = _repack(wt3)
    wm, bv = _sc_call(X.T, I.T, S.T, wlin, bias)
    out_t = _tc_call(wm.T, bv)
    return out_t.transpose(0, 2, 1)
